# baseline (speedup 1.0000x reference)
import functools

import jax
import jax.numpy as jnp
from jax import lax
from jax.experimental import pallas as pl

V = 16
D = 16
BB = 128
CH = BB * 200


def _table_body(emb_ref, w_ref, b_ref, out_ref):
    out_ref[...] = lax.dot_general(
        emb_ref[...], w_ref[...],
        dimension_numbers=(((1,), (1,)), ((), ())),
        preferred_element_type=jnp.float32,
    ) + b_ref[...]


def _build_table(emb, W, b):
    b2 = jnp.broadcast_to(b[None, :], (V, V))
    return pl.pallas_call(
        _table_body,
        out_shape=jax.ShapeDtypeStruct((V, V), jnp.float32),
    )(emb, W, b2)


def _onehot_body(ids_ref, table_ref, o_ref):
    ids = ids_ref[...]
    oh = (jnp.broadcast_to(ids[None, :], (V, CH))
          == lax.broadcasted_iota(jnp.int32, (V, CH), 0)).astype(jnp.float32)
    rows = lax.dot_general(
        oh, table_ref[...],
        dimension_numbers=(((0,), (0,)), ((), ())),
        preferred_element_type=jnp.float32,
    )
    o_ref[...] = rows.reshape(BB, CH // BB, V)


@functools.lru_cache(maxsize=None)
def _make_lookup(batch: int, seq: int):
    assert batch % BB == 0 and BB * seq == CH
    return pl.pallas_call(
        _onehot_body,
        grid=(batch // BB,),
        in_specs=[
            pl.BlockSpec((CH,), lambda i: (i,)),
            pl.BlockSpec((V, V), lambda i: (0, 0)),
        ],
        out_specs=pl.BlockSpec((BB, seq, V), lambda i: (i, 0, 0)),
        out_shape=jax.ShapeDtypeStruct((batch, seq, V), jnp.float32),
    )


def kernel(input_ids, emb, W, b):
    batch, seq = input_ids.shape
    ids = input_ids.reshape(batch * seq).astype(jnp.int32)
    table = _build_table(emb, W, b)
    return _make_lookup(batch, seq)(ids, table)
